# baseline (speedup 1.0000x reference)
import functools

import jax
import jax.numpy as jnp
from jax import lax
from jax.experimental import pallas as pl
from jax.experimental.pallas import tpu as pltpu
from jax.experimental.pallas import tpu_sc as plsc

_SMOOTH = 0.1
_ROW_BLOCK = 8
_NC = 2
_NS = 16
_LANES = 16


def _sc_gather_body(pred_hbm, tgt_hbm, out_hbm, idx_v, vals_v, sem, *,
                    num_classes, bpw):
    wid = lax.axis_index("s") * _NC + lax.axis_index("c")
    base = wid * bpw
    pltpu.sync_copy(tgt_hbm.at[pl.ds(base, bpw)], idx_v)
    for j in range(bpw // _LANES):
        t16 = idx_v[pl.ds(j * _LANES, _LANES)]
        rows = lax.iota(jnp.int32, _LANES) + (base + j * _LANES)
        idx_v[pl.ds(j * _LANES, _LANES)] = rows * num_classes + t16
    pltpu.async_copy(pred_hbm.at[idx_v], vals_v, sem).wait()
    pltpu.sync_copy(vals_v, out_hbm.at[pl.ds(base, bpw)])


def _sc_gather(pred_flat, tgt):
    batch = tgt.shape[0]
    num_classes = pred_flat.shape[0] // batch
    bpw = batch // (_NC * _NS)
    mesh = plsc.VectorSubcoreMesh(core_axis_name="c", subcore_axis_name="s")
    k = functools.partial(
        pl.kernel,
        out_type=jax.ShapeDtypeStruct((batch,), jnp.float32),
        mesh=mesh,
        scratch_types=[
            pltpu.VMEM((bpw,), jnp.int32),
            pltpu.VMEM((bpw,), jnp.float32),
            pltpu.SemaphoreType.DMA,
        ],
    )(functools.partial(_sc_gather_body, num_classes=num_classes, bpw=bpw))
    return k(pred_flat, tgt)


def _loss_kernel(pred_ref, pt_ref, out_ref, *, num_classes, batch):
    x = pred_ref[...]
    pt = pt_ref[...]

    m = jnp.max(x, axis=1, keepdims=True)
    s_sum = jnp.sum(x, axis=1, keepdims=True)
    e_sum = jnp.sum(jnp.exp(x - m), axis=1, keepdims=True)

    lse = jnp.log(e_sum)
    eps = _SMOOTH / (num_classes - 1)
    row_loss = -(
        eps * (s_sum - num_classes * (m + lse))
        + (1.0 - _SMOOTH - eps) * (pt - m - lse)
    )

    @pl.when(pl.program_id(0) == 0)
    def _():
        out_ref[...] = jnp.zeros((1, 1), jnp.float32)

    out_ref[...] += jnp.sum(row_loss).reshape(1, 1) / batch


def kernel(pred, target):
    batch, num_classes = pred.shape
    tgt = target.astype(jnp.int32)

    pt = _sc_gather(pred.reshape(-1), tgt).reshape(batch, 1)

    grid = batch // _ROW_BLOCK
    out = pl.pallas_call(
        functools.partial(_loss_kernel, num_classes=num_classes, batch=batch),
        grid=(grid,),
        in_specs=[
            pl.BlockSpec((_ROW_BLOCK, num_classes), lambda i: (i, 0)),
            pl.BlockSpec((_ROW_BLOCK, 1), lambda i: (i, 0)),
        ],
        out_specs=pl.BlockSpec((1, 1), lambda i: (0, 0)),
        out_shape=jax.ShapeDtypeStruct((1, 1), jnp.float32),
    )(pred, pt)
    return out[0, 0]


# speedup vs baseline: 1.8470x; 1.8470x over previous
import functools

import jax
import jax.numpy as jnp
from jax.experimental import pallas as pl
from jax.experimental.pallas import tpu as pltpu

_SMOOTH = 0.1
_ROW_BLOCK = 8
_LANE = 128


def _loss_kernel(tgt_smem, pred_ref, *strips_out, num_classes, batch):
    strips = strips_out[:-1]
    out_ref = strips_out[-1]
    i = pl.program_id(0)
    rb = pred_ref.shape[0]

    x = pred_ref[...]
    m = jnp.max(x, axis=1, keepdims=True)
    s_sum = jnp.sum(x, axis=1, keepdims=True)
    e_sum = jnp.sum(jnp.exp(x - m), axis=1, keepdims=True)
    lse = jnp.log(e_sum)

    eps = _SMOOTH / (num_classes - 1)
    coef = 1.0 - _SMOOTH - eps
    vec_part = -(
        eps * (s_sum - num_classes * (m + lse)) + coef * (-m - lse)
    )

    lane = jax.lax.broadcasted_iota(jnp.int32, (1, _LANE), 1)
    pt_total = 0.0
    for j in range(rb):
        t = tgt_smem[i * rb + j]
        off = jax.lax.rem(t, _LANE)
        row = strips[j][j % 8, :].reshape(1, _LANE)
        pt_total += jnp.sum(jnp.where(lane == off, row, 0.0))

    block_sum = jnp.sum(vec_part) - coef * pt_total

    @pl.when(i == 0)
    def _():
        out_ref[...] = jnp.zeros((1, 1), jnp.float32)

    out_ref[...] += block_sum.reshape(1, 1) / batch


def _strip_spec(j, rb):
    def index_map(i, tref):
        r = i * rb + j
        return (r // 8, tref[r] // _LANE)

    return pl.BlockSpec((8, _LANE), index_map)


def kernel(pred, target):
    batch, num_classes = pred.shape
    tgt = target.astype(jnp.int32)
    rb = _ROW_BLOCK
    grid = batch // rb

    grid_spec = pltpu.PrefetchScalarGridSpec(
        num_scalar_prefetch=1,
        grid=(grid,),
        in_specs=[
            pl.BlockSpec((rb, num_classes), lambda i, tref: (i, 0)),
            *[_strip_spec(j, rb) for j in range(rb)],
        ],
        out_specs=pl.BlockSpec((1, 1), lambda i, tref: (0, 0)),
    )
    out = pl.pallas_call(
        functools.partial(_loss_kernel, num_classes=num_classes, batch=batch),
        grid_spec=grid_spec,
        out_shape=jax.ShapeDtypeStruct((1, 1), jnp.float32),
    )(tgt, pred, *([pred] * rb))
    return out[0, 0]


# speedup vs baseline: 2.3158x; 1.2538x over previous
import functools

import jax
import jax.numpy as jnp
from jax.experimental import pallas as pl
from jax.experimental.pallas import tpu as pltpu

_SMOOTH = 0.1
_ROW_BLOCK = 32
_LANE = 128


def _loss_kernel(tgt_smem, pred_ref, *strips_out, num_classes, batch):
    strips = strips_out[:-1]
    out_ref = strips_out[-1]
    i = pl.program_id(0)
    rb = pred_ref.shape[0]

    x = pred_ref[...]
    m = jnp.max(x, axis=1, keepdims=True)
    s_sum = jnp.sum(x, axis=1, keepdims=True)
    e_sum = jnp.sum(jnp.exp(x - m), axis=1, keepdims=True)
    lse = jnp.log(e_sum)

    eps = _SMOOTH / (num_classes - 1)
    coef = 1.0 - _SMOOTH - eps
    vec_part = -(
        eps * (s_sum - num_classes * (m + lse)) + coef * (-m - lse)
    )

    lane = jax.lax.broadcasted_iota(jnp.int32, (1, _LANE), 1)
    pt_total = 0.0
    for j in range(rb):
        t = tgt_smem[i * rb + j]
        off = jax.lax.rem(t, _LANE)
        row = strips[j][j % 8, :].reshape(1, _LANE)
        pt_total += jnp.sum(jnp.where(lane == off, row, 0.0))

    block_sum = jnp.sum(vec_part) - coef * pt_total

    @pl.when(i == 0)
    def _():
        out_ref[...] = jnp.zeros((1, 1), jnp.float32)

    out_ref[...] += block_sum.reshape(1, 1) / batch


def _strip_spec(j, rb):
    def index_map(i, tref):
        r = i * rb + j
        return (r // 8, tref[r] // _LANE)

    return pl.BlockSpec((8, _LANE), index_map)


def kernel(pred, target):
    batch, num_classes = pred.shape
    tgt = target.astype(jnp.int32)
    rb = _ROW_BLOCK
    grid = batch // rb

    grid_spec = pltpu.PrefetchScalarGridSpec(
        num_scalar_prefetch=1,
        grid=(grid,),
        in_specs=[
            pl.BlockSpec((rb, num_classes), lambda i, tref: (i, 0)),
            *[_strip_spec(j, rb) for j in range(rb)],
        ],
        out_specs=pl.BlockSpec((1, 1), lambda i, tref: (0, 0)),
    )
    out = pl.pallas_call(
        functools.partial(_loss_kernel, num_classes=num_classes, batch=batch),
        grid_spec=grid_spec,
        out_shape=jax.ShapeDtypeStruct((1, 1), jnp.float32),
    )(tgt, pred, *([pred] * rb))
    return out[0, 0]
